# baseline (speedup 1.0000x reference)
import functools
import jax
import jax.numpy as jnp
from jax import lax
from jax.experimental import pallas as pl
from jax.experimental.pallas import tpu as pltpu
from jax.experimental.pallas import tpu_sc as plsc

_N = 2048
_F = 16
_BLK = 512
_NW = 32
_W = 16


def _sc_b_kernel(e_hbm, dm_hbm, out_hbm, e_v, dm_v, p_v, sem):
    sc_rows = e_hbm.shape[0]
    rpw = sc_rows // _NW
    c = lax.axis_index("c")
    s = lax.axis_index("s")
    wid = s * 2 + c
    base = wid * rpw

    def batch_body(bt, _):
        row0 = base + bt * _W
        pltpu.async_copy(e_hbm.at[pl.ds(row0, _W), :], e_v, sem).wait()
        pltpu.async_copy(dm_hbm.at[pl.ds(row0, _W), :, :], dm_v, sem).wait()

        def row_body(r, _2):
            def j_body(jc, accs):
                a0, a1 = accs
                ve = e_v[r, pl.ds(jc * 16, 16)]
                vd0 = dm_v[r, 0, pl.ds(jc * 16, 16)]
                vd1 = dm_v[r, 1, pl.ds(jc * 16, 16)]
                return (a0 + ve * vd0, a1 + ve * vd1)

            z = jnp.zeros((16,), jnp.float32)
            a0, a1 = lax.fori_loop(0, _N // 16, j_body, (z, z))
            p_v[0, r, :] = a0
            p_v[1, r, :] = a1
            return _2

        lax.fori_loop(0, _W, row_body, 0)
        pltpu.sync_copy(p_v, out_hbm.at[:, pl.ds(row0, _W), :])
        return _

    lax.fori_loop(0, rpw // _W, batch_body, 0)


def _sc_b(edges_part, dmt_part):
    sc_rows = edges_part.shape[0]
    kfn = pl.kernel(
        _sc_b_kernel,
        out_type=jax.ShapeDtypeStruct((2, sc_rows, 16), jnp.float32),
        mesh=plsc.VectorSubcoreMesh(core_axis_name="c", subcore_axis_name="s"),
        scratch_types=[
            pltpu.VMEM((_W, _N), jnp.float32),
            pltpu.VMEM((_W, 2, _N), jnp.float32),
            pltpu.VMEM((2, _W, 16), jnp.float32),
            pltpu.SemaphoreType.DMA,
        ],
    )
    return kfn(edges_part, dmt_part)


def _tc_main_block(x_ref, e_ref, w1x_ref, w2_ref, o_ref):
    i = pl.program_id(0)
    x = x_ref[...]
    e = e_ref[...]
    xw = jnp.dot(x, w1x_ref[...], preferred_element_type=jnp.float32)
    agg = jnp.dot(e, xw, preferred_element_type=jnp.float32)
    xi = x_ref[pl.ds(i * _BLK, _BLK), :]
    o_ref[...] = jnp.dot(xi, w2_ref[...], preferred_element_type=jnp.float32) + agg


def _tc_combine_block(m_ref, p_ref, w1d_ref, o_ref):
    p0 = p_ref[0]
    p1 = p_ref[1]
    b0 = jnp.sum(p0, axis=1, keepdims=True)
    b1 = jnp.sum(p1, axis=1, keepdims=True)
    w1d = w1d_ref[...]
    bc = b0 * w1d[0, :][None, :] + b1 * w1d[1, :][None, :]
    o_ref[...] = m_ref[...] + bc


def kernel(x, edges, distance_matrix, w1, w2):
    w1x = w1[:, :_F].T
    w1d = jnp.zeros((8, _F), jnp.float32).at[:2].set(w1[:, _F:].T)
    w2t = w2.T
    dmt = jnp.transpose(distance_matrix, (0, 2, 1))

    pmat = _sc_b(edges, dmt)

    main = pl.pallas_call(
        _tc_main_block,
        grid=(_N // _BLK,),
        in_specs=[
            pl.BlockSpec((_N, _F), lambda i: (0, 0)),
            pl.BlockSpec((_BLK, _N), lambda i: (i, 0)),
            pl.BlockSpec((_F, _F), lambda i: (0, 0)),
            pl.BlockSpec((_F, _F), lambda i: (0, 0)),
        ],
        out_specs=pl.BlockSpec((_BLK, _F), lambda i: (i, 0)),
        out_shape=jax.ShapeDtypeStruct((_N, _F), jnp.float32),
    )(x, edges, w1x, w2t)

    return pl.pallas_call(
        _tc_combine_block,
        in_specs=[
            pl.BlockSpec((_N, _F), lambda i: (0, 0)),
            pl.BlockSpec((2, _N, _F), lambda i: (0, 0, 0)),
            pl.BlockSpec((8, _F), lambda i: (0, 0)),
        ],
        grid=(1,),
        out_specs=pl.BlockSpec((_N, _F), lambda i: (0, 0)),
        out_shape=jax.ShapeDtypeStruct((_N, _F), jnp.float32),
    )(main, pmat, w1d)


# speedup vs baseline: 1.6295x; 1.6295x over previous
import jax
import jax.numpy as jnp
from jax import lax
from jax.experimental import pallas as pl
from jax.experimental.pallas import tpu as pltpu
from jax.experimental.pallas import tpu_sc as plsc

_N = 2048
_F = 16
_BLK = 256
_NW = 32
_W = 8
_SCR = 768


def _sc_b_kernel(e_hbm, dm_hbm, out_hbm, e_v, dm_v, p_v, sem):
    rpw = _SCR // _NW
    c = lax.axis_index("c")
    s = lax.axis_index("s")
    wid = s * 2 + c
    base = wid * rpw

    def batch_body(bt, _):
        row0 = base + bt * _W
        pltpu.async_copy(e_hbm.at[pl.ds(row0, _W), :], e_v, sem).wait()
        pltpu.async_copy(dm_hbm.at[pl.ds(row0, _W), :, :], dm_v, sem).wait()

        def row_body(r, _2):
            def j_body(jc, accs):
                a0, a1 = accs
                ve = e_v[r, pl.ds(jc * 16, 16)]
                vd0 = dm_v[r, 0, pl.ds(jc * 16, 16)]
                vd1 = dm_v[r, 1, pl.ds(jc * 16, 16)]
                return (a0 + ve * vd0, a1 + ve * vd1)

            z = jnp.zeros((16,), jnp.float32)
            a0, a1 = lax.fori_loop(0, _N // 16, j_body, (z, z))
            p_v[0, r, :] = a0
            p_v[1, r, :] = a1
            return _2

        lax.fori_loop(0, _W, row_body, 0)
        pltpu.sync_copy(p_v, out_hbm.at[:, pl.ds(row0, _W), :])
        return _

    lax.fori_loop(0, rpw // _W, batch_body, 0)


def _sc_b(edges_part, dmt_part):
    kfn = pl.kernel(
        _sc_b_kernel,
        out_type=jax.ShapeDtypeStruct((2, _SCR, 16), jnp.float32),
        mesh=plsc.VectorSubcoreMesh(core_axis_name="c", subcore_axis_name="s"),
        scratch_types=[
            pltpu.VMEM((_W, _N), jnp.float32),
            pltpu.VMEM((_W, 2, _N), jnp.float32),
            pltpu.VMEM((2, _W, 16), jnp.float32),
            pltpu.SemaphoreType.DMA,
        ],
    )
    return kfn(edges_part, dmt_part)


def _tc_main_block(x_ref, e_ref, w1x_ref, w2_ref, o_ref):
    i = pl.program_id(0)
    x = x_ref[...]
    e = e_ref[...]
    xw = jnp.dot(x, w1x_ref[...], preferred_element_type=jnp.float32)
    agg = jnp.dot(e, xw, preferred_element_type=jnp.float32)
    xi = x_ref[pl.ds(i * _BLK, _BLK), :]
    o_ref[...] = jnp.dot(xi, w2_ref[...], preferred_element_type=jnp.float32) + agg


def _tc_full_block(x_ref, e_ref, dm_ref, w1x_ref, w1d_ref, w2_ref, o_ref):
    i = pl.program_id(0)
    x = x_ref[...]
    e = e_ref[...]
    xw = jnp.dot(x, w1x_ref[...], preferred_element_type=jnp.float32)
    agg = jnp.dot(e, xw, preferred_element_type=jnp.float32)
    d0 = dm_ref[:, 0, :]
    d1 = dm_ref[:, 1, :]
    b0 = jnp.sum(e * d0, axis=1, keepdims=True)
    b1 = jnp.sum(e * d1, axis=1, keepdims=True)
    w1d = w1d_ref[...]
    bc = b0 * w1d[0, :][None, :] + b1 * w1d[1, :][None, :]
    xi = x_ref[pl.ds(_SCR + i * _BLK, _BLK), :]
    o_ref[...] = (
        jnp.dot(xi, w2_ref[...], preferred_element_type=jnp.float32) + agg + bc
    )


def _tc_combine_block(m_ref, p_ref, w1d_ref, o_ref):
    p0 = p_ref[0]
    p1 = p_ref[1]
    b0 = jnp.sum(p0, axis=1, keepdims=True)
    b1 = jnp.sum(p1, axis=1, keepdims=True)
    w1d = w1d_ref[...]
    bc = b0 * w1d[0, :][None, :] + b1 * w1d[1, :][None, :]
    o_ref[...] = m_ref[...] + bc


def kernel(x, edges, distance_matrix, w1, w2):
    w1x = w1[:, :_F].T
    w1d = jnp.zeros((8, _F), jnp.float32).at[:2].set(w1[:, _F:].T)
    w2t = w2.T
    dmt = jnp.transpose(distance_matrix, (0, 2, 1))

    pmat = _sc_b(edges, dmt)

    main_a = pl.pallas_call(
        _tc_main_block,
        grid=(_SCR // _BLK,),
        in_specs=[
            pl.BlockSpec((_N, _F), lambda i: (0, 0)),
            pl.BlockSpec((_BLK, _N), lambda i: (i, 0)),
            pl.BlockSpec((_F, _F), lambda i: (0, 0)),
            pl.BlockSpec((_F, _F), lambda i: (0, 0)),
        ],
        out_specs=pl.BlockSpec((_BLK, _F), lambda i: (i, 0)),
        out_shape=jax.ShapeDtypeStruct((_SCR, _F), jnp.float32),
    )(x, edges, w1x, w2t)

    nb = _SCR // _BLK
    out_b = pl.pallas_call(
        _tc_full_block,
        grid=((_N - _SCR) // _BLK,),
        in_specs=[
            pl.BlockSpec((_N, _F), lambda i: (0, 0)),
            pl.BlockSpec((_BLK, _N), lambda i: (i + nb, 0)),
            pl.BlockSpec((_BLK, 2, _N), lambda i: (i + nb, 0, 0)),
            pl.BlockSpec((_F, _F), lambda i: (0, 0)),
            pl.BlockSpec((8, _F), lambda i: (0, 0)),
            pl.BlockSpec((_F, _F), lambda i: (0, 0)),
        ],
        out_specs=pl.BlockSpec((_BLK, _F), lambda i: (i, 0)),
        out_shape=jax.ShapeDtypeStruct((_N - _SCR, _F), jnp.float32),
    )(x, edges, dmt, w1x, w1d, w2t)

    out_a = pl.pallas_call(
        _tc_combine_block,
        grid=(1,),
        in_specs=[
            pl.BlockSpec((_SCR, _F), lambda i: (0, 0)),
            pl.BlockSpec((2, _SCR, _F), lambda i: (0, 0, 0)),
            pl.BlockSpec((8, _F), lambda i: (0, 0)),
        ],
        out_specs=pl.BlockSpec((_SCR, _F), lambda i: (0, 0)),
        out_shape=jax.ShapeDtypeStruct((_SCR, _F), jnp.float32),
    )(main_a, pmat, w1d)

    return jnp.concatenate([out_a, out_b], axis=0)


# speedup vs baseline: 2.7286x; 1.6745x over previous
import jax
import jax.numpy as jnp
from jax.experimental import pallas as pl

_N = 2048
_F = 16
_BLK = 512


def _gcn_block(x_ref, e_ref, dm_ref, w1x_ref, w1d_ref, w2_ref, o_ref):
    i = pl.program_id(0)
    x = x_ref[...]
    e = e_ref[...]
    xw = jnp.dot(x, w1x_ref[...], preferred_element_type=jnp.float32)
    agg = jnp.dot(e, xw, preferred_element_type=jnp.float32)
    d0 = dm_ref[:, 0, :]
    d1 = dm_ref[:, 1, :]
    b0 = jnp.sum(e * d0, axis=1, keepdims=True)
    b1 = jnp.sum(e * d1, axis=1, keepdims=True)
    w1d = w1d_ref[...]
    bc = b0 * w1d[0, :][None, :] + b1 * w1d[1, :][None, :]
    xi = x_ref[pl.ds(i * _BLK, _BLK), :]
    o_ref[...] = (
        jnp.dot(xi, w2_ref[...], preferred_element_type=jnp.float32) + agg + bc
    )


def kernel(x, edges, distance_matrix, w1, w2):
    w1x = w1[:, :_F].T
    w1d = jnp.zeros((8, _F), jnp.float32).at[:2].set(w1[:, _F:].T)
    w2t = w2.T
    dmt = jnp.transpose(distance_matrix, (0, 2, 1))

    grid = (_N // _BLK,)
    return pl.pallas_call(
        _gcn_block,
        grid=grid,
        in_specs=[
            pl.BlockSpec((_N, _F), lambda i: (0, 0)),
            pl.BlockSpec((_BLK, _N), lambda i: (i, 0)),
            pl.BlockSpec((_BLK, 2, _N), lambda i: (i, 0, 0)),
            pl.BlockSpec((_F, _F), lambda i: (0, 0)),
            pl.BlockSpec((8, _F), lambda i: (0, 0)),
            pl.BlockSpec((_F, _F), lambda i: (0, 0)),
        ],
        out_specs=pl.BlockSpec((_BLK, _F), lambda i: (i, 0)),
        out_shape=jax.ShapeDtypeStruct((_N, _F), jnp.float32),
    )(x, edges, dmt, w1x, w1d, w2t)
